# baseline (speedup 1.0000x reference)
import functools
import math

import jax
import jax.numpy as jnp
import numpy as np
from jax import lax
from jax.experimental import pallas as pl
from jax.experimental.pallas import tpu as pltpu
from jax.experimental.pallas import tpu_sc as plsc

_VOCAB = 100000
_D = 1024
_B = 4
_S = 2048
_SCALE = math.sqrt(_D)

_NW = 32
_P_PER_W = _S // _NW
_PC = 16
_NCH = _P_PER_W // _PC
_H = _D // 2
_NITEM = _NCH * 2
_NSLOT = 3
_LANES = 16


def _sin_pe(max_len, d_model):
    pos = np.arange(max_len, dtype=np.float32)[:, None]
    div = np.exp(
        np.arange(0, d_model, 2, dtype=np.float32) * (-math.log(10000.0) / d_model)
    )
    pe = np.zeros((max_len, d_model), dtype=np.float32)
    pe[:, 0::2] = np.sin(pos * div)
    pe[:, 1::2] = np.cos(pos * div)
    return pe


_PE = _sin_pe(_S, _D)

_mesh = plsc.VectorSubcoreMesh(core_axis_name="c", subcore_axis_name="s")

_row_buf_types = [
    pltpu.VMEM((_PC, _H), jnp.float32) for _ in range(_NSLOT * _B)
]


@functools.partial(
    pl.kernel,
    mesh=_mesh,
    out_type=jax.ShapeDtypeStruct((_B * _S, 2, _H), jnp.float32),
    scratch_types=[
        pltpu.VMEM((_B, _P_PER_W), jnp.int32),
        pltpu.VMEM((2, _B, _P_PER_W), jnp.int32),
        *_row_buf_types,
        pltpu.VMEM((_PC, _H), jnp.float32),
        pltpu.VMEM((_PC, _H), jnp.float32),
        pltpu.SemaphoreType.DMA,
        pltpu.SemaphoreType.DMA,
        pltpu.SemaphoreType.DMA,
        pltpu.SemaphoreType.DMA,
        pltpu.SemaphoreType.DMA,
        pltpu.SemaphoreType.DMA,
        pltpu.SemaphoreType.DMA,
        pltpu.SemaphoreType.DMA,
    ],
)
def _emb_kernel(ids_hbm, w_hbm, pe0_hbm, pe1_hbm, out_hbm,
                idx_v, idx2_v, *bufs_and_sems):
    slots = [bufs_and_sems[s * _B:(s + 1) * _B] for s in range(_NSLOT)]
    pes = bufs_and_sems[_NSLOT * _B:_NSLOT * _B + 2]
    gsems = bufs_and_sems[_NSLOT * _B + 2:_NSLOT * _B + 5]
    ssems = bufs_and_sems[_NSLOT * _B + 5:_NSLOT * _B + 8]
    psems = bufs_and_sems[_NSLOT * _B + 8:_NSLOT * _B + 10]
    pe_hbms = (pe0_hbm, pe1_hbm)

    wid = lax.axis_index("s") * 2 + lax.axis_index("c")
    base_p = wid * _P_PER_W

    def gather_copies(i):
        c, h = i // 2, i % 2
        s = i % _NSLOT
        return [
            pltpu.make_async_copy(
                w_hbm.at[idx2_v.at[h, b, pl.ds(c * _PC, _PC)]],
                slots[s][b], gsems[s])
            for b in range(_B)
        ]

    def store_copies(i):
        c, h = i // 2, i % 2
        s = i % _NSLOT
        return [
            pltpu.make_async_copy(
                slots[s][b],
                out_hbm.at[pl.ds(b * _S + base_p + c * _PC, _PC), h],
                ssems[s])
            for b in range(_B)
        ]

    def pe_copy(i):
        c, h = i // 2, i % 2
        return pltpu.make_async_copy(
            pe_hbms[h].at[pl.ds(base_p + c * _PC, _PC)], pes[i % 2], psems[i % 2])

    for b in range(_B):
        pltpu.sync_copy(ids_hbm.at[pl.ds(b * _S + base_p, _P_PER_W)],
                        idx_v.at[b])
    for b in range(_B):
        for k in range(_P_PER_W // _LANES):
            sl = pl.ds(k * _LANES, _LANES)
            v2 = idx_v[b, sl] * 2
            idx2_v[0, b, sl] = v2
            idx2_v[1, b, sl] = v2 + 1

    pe_copy(0).start()
    for cp in gather_copies(0):
        cp.start()
    for cp in gather_copies(1):
        cp.start()

    for i in range(_NITEM):
        if i + 1 < _NITEM:
            pe_copy(i + 1).start()
        if i + 2 < _NITEM:
            if i >= 1:
                for cp in store_copies(i - 1):
                    cp.wait()
            for cp in gather_copies(i + 2):
                cp.start()
        pe_copy(i).wait()
        for cp in gather_copies(i):
            cp.wait()

        rbs = slots[i % _NSLOT]
        pb = pes[i % 2]

        def body_r(r, _):
            def body_j(j, _):
                for jj in range(4):
                    sl = pl.ds((j * 4 + jj) * _LANES, _LANES)
                    pv = pb[r, sl]
                    for rb in rbs:
                        rb[r, sl] = rb[r, sl] * _SCALE + pv
                return 0

            return lax.fori_loop(0, _H // (_LANES * 4), body_j, 0)

        lax.fori_loop(0, _PC, body_r, 0)
        for cp in store_copies(i):
            cp.start()

    for i in range(_NITEM - 3, _NITEM):
        for cp in store_copies(i):
            cp.wait()


def kernel(token_ids, W):
    ids = token_ids.reshape(-1).astype(jnp.int32)
    w2 = W.reshape(2 * _VOCAB, _H)
    pe0 = jnp.asarray(_PE[:, :_H])
    pe1 = jnp.asarray(_PE[:, _H:])
    out = _emb_kernel(ids, w2, pe0, pe1)
    return out.reshape(_B, _S, _D)


# speedup vs baseline: 6.5420x; 6.5420x over previous
import functools
import math

import jax
import jax.numpy as jnp
import numpy as np
from jax import lax
from jax.experimental import pallas as pl
from jax.experimental.pallas import tpu as pltpu
from jax.experimental.pallas import tpu_sc as plsc

_VOCAB = 100000
_D = 1024
_B = 4
_S = 2048
_SCALE = math.sqrt(_D)

_NW = 32
_P_PER_W = _S // _NW
_PC = 8
_NITEM = _P_PER_W // _PC
_NSLOT = 3
_LANES = 16


def _sin_pe(max_len, d_model):
    pos = np.arange(max_len, dtype=np.float32)[:, None]
    div = np.exp(
        np.arange(0, d_model, 2, dtype=np.float32) * (-math.log(10000.0) / d_model)
    )
    pe = np.zeros((max_len, d_model), dtype=np.float32)
    pe[:, 0::2] = np.sin(pos * div)
    pe[:, 1::2] = np.cos(pos * div)
    return pe


_PE = _sin_pe(_S, _D)

_mesh = plsc.VectorSubcoreMesh(core_axis_name="c", subcore_axis_name="s")

_row_buf_types = [
    pltpu.VMEM((_PC, _D), jnp.float32) for _ in range(_NSLOT * _B)
]


@functools.partial(
    pl.kernel,
    mesh=_mesh,
    out_type=jax.ShapeDtypeStruct((_B * _S, _D), jnp.float32),
    scratch_types=[
        pltpu.VMEM((_B, _P_PER_W), jnp.int32),
        *_row_buf_types,
        pltpu.VMEM((_PC, _D), jnp.float32),
        pltpu.VMEM((_PC, _D), jnp.float32),
        pltpu.SemaphoreType.DMA,
        pltpu.SemaphoreType.DMA,
        pltpu.SemaphoreType.DMA,
        pltpu.SemaphoreType.DMA,
        pltpu.SemaphoreType.DMA,
        pltpu.SemaphoreType.DMA,
        pltpu.SemaphoreType.DMA,
        pltpu.SemaphoreType.DMA,
    ],
)
def _emb_kernel(ids_hbm, w_hbm, pe_hbm, out_hbm,
                idx_v, *bufs_and_sems):
    slots = [bufs_and_sems[s * _B:(s + 1) * _B] for s in range(_NSLOT)]
    k = _NSLOT * _B
    pes = bufs_and_sems[k:k + 2]
    gsems = bufs_and_sems[k + 2:k + 5]
    ssems = bufs_and_sems[k + 5:k + 8]
    psems = bufs_and_sems[k + 8:k + 10]

    wid = lax.axis_index("s") * 2 + lax.axis_index("c")
    base_p = wid * _P_PER_W

    def gather_copies(i):
        s = i % _NSLOT
        return [
            pltpu.make_async_copy(
                w_hbm.at[idx_v.at[b, pl.ds(i * _PC, _PC)]],
                slots[s][b], gsems[s])
            for b in range(_B)
        ]

    def store_copies(i):
        s = i % _NSLOT
        return [
            pltpu.make_async_copy(
                slots[s][b],
                out_hbm.at[pl.ds(b * _S + base_p + i * _PC, _PC)],
                ssems[s])
            for b in range(_B)
        ]

    def pe_copy(i):
        return pltpu.make_async_copy(
            pe_hbm.at[pl.ds(base_p + i * _PC, _PC)], pes[i % 2], psems[i % 2])

    for b in range(_B):
        pltpu.sync_copy(ids_hbm.at[pl.ds(b * _S + base_p, _P_PER_W)],
                        idx_v.at[b])
    pe_copy(0).start()
    for cp in gather_copies(0):
        cp.start()
    for cp in gather_copies(1):
        cp.start()

    for i in range(_NITEM):
        if i + 1 < _NITEM:
            pe_copy(i + 1).start()
        if i + 2 < _NITEM:
            if i >= 1:
                for cp in store_copies(i - 1):
                    cp.wait()
            for cp in gather_copies(i + 2):
                cp.start()
        pe_copy(i).wait()
        for cp in gather_copies(i):
            cp.wait()

        rbs = slots[i % _NSLOT]
        pb = pes[i % 2]

        def body_r(r, _):
            def body_j(j, _):
                for jj in range(4):
                    sl = pl.ds((j * 4 + jj) * _LANES, _LANES)
                    pv = pb[r, sl]
                    for rb in rbs:
                        rb[r, sl] = rb[r, sl] * _SCALE + pv
                return 0

            return lax.fori_loop(0, _D // (_LANES * 4), body_j, 0)

        lax.fori_loop(0, _PC, body_r, 0)
        for cp in store_copies(i):
            cp.start()

    for i in range(_NITEM - 3, _NITEM):
        for cp in store_copies(i):
            cp.wait()


def kernel(token_ids, W):
    ids = token_ids.reshape(-1).astype(jnp.int32)
    pe = jnp.asarray(_PE)
    out = _emb_kernel(ids, W, pe)
    return out.reshape(_B, _S, _D)


# speedup vs baseline: 8.6554x; 1.3230x over previous
import functools
import math

import jax
import jax.numpy as jnp
import numpy as np
from jax import lax
from jax.experimental import pallas as pl
from jax.experimental.pallas import tpu as pltpu
from jax.experimental.pallas import tpu_sc as plsc

_VOCAB = 100000
_D = 1024
_B = 4
_S = 2048
_SCALE = math.sqrt(_D)

_NW = 32
_P_PER_W = _S // _NW
_PC = 16
_NCH = _P_PER_W // _PC
_NITEM = _NCH * _B
_LANES = 16


def _sin_pe(max_len, d_model):
    pos = np.arange(max_len, dtype=np.float32)[:, None]
    div = np.exp(
        np.arange(0, d_model, 2, dtype=np.float32) * (-math.log(10000.0) / d_model)
    )
    pe = np.zeros((max_len, d_model), dtype=np.float32)
    pe[:, 0::2] = np.sin(pos * div)
    pe[:, 1::2] = np.cos(pos * div)
    return pe


_PE = _sin_pe(_S, _D)

_mesh = plsc.VectorSubcoreMesh(core_axis_name="c", subcore_axis_name="s")


@functools.partial(
    pl.kernel,
    mesh=_mesh,
    out_type=jax.ShapeDtypeStruct((_B, _S, _D), jnp.float32),
    scratch_types=[
        pltpu.VMEM((_B, _P_PER_W), jnp.int32),
        pltpu.VMEM((_PC, _D), jnp.float32),
        pltpu.VMEM((_PC, _D), jnp.float32),
        pltpu.VMEM((_PC, _D), jnp.float32),
        pltpu.VMEM((_PC, _D), jnp.float32),
        pltpu.VMEM((_PC, _D), jnp.float32),
        pltpu.VMEM((_PC, _D), jnp.float32),
        pltpu.SemaphoreType.DMA,
        pltpu.SemaphoreType.DMA,
        pltpu.SemaphoreType.DMA,
        pltpu.SemaphoreType.DMA,
        pltpu.SemaphoreType.DMA,
        pltpu.SemaphoreType.DMA,
        pltpu.SemaphoreType.DMA,
        pltpu.SemaphoreType.DMA,
        pltpu.SemaphoreType.DMA,
        pltpu.SemaphoreType.DMA,
    ],
)
def _emb_kernel(ids_hbm, w_hbm, pe_hbm, out_hbm,
                idx_v, r0, r1, r2, r3, pe0, pe1,
                g0, g1, g2, g3, s0, s1, s2, s3, psem0, psem1):
    rows = (r0, r1, r2, r3)
    pes = (pe0, pe1)
    gsems = (g0, g1, g2, g3)
    ssems = (s0, s1, s2, s3)
    psems = (psem0, psem1)

    wid = lax.axis_index("s") * 2 + lax.axis_index("c")
    base_p = wid * _P_PER_W

    def gather_copy(i):
        b, c = i % _B, i // _B
        return pltpu.make_async_copy(
            w_hbm.at[idx_v.at[b, pl.ds(c * _PC, _PC)]], rows[b], gsems[b])

    def store_copy(i):
        b, c = i % _B, i // _B
        return pltpu.make_async_copy(
            rows[b], out_hbm.at[b, pl.ds(base_p + c * _PC, _PC)], ssems[b])

    def pe_copy(c):
        return pltpu.make_async_copy(
            pe_hbm.at[pl.ds(base_p + c * _PC, _PC)], pes[c % 2], psems[c % 2])

    for b in range(_B):
        pltpu.sync_copy(ids_hbm.at[b, pl.ds(base_p, _P_PER_W)], idx_v.at[b])
    pe_copy(0).start()
    gather_copy(0).start()
    gather_copy(1).start()

    for i in range(_NITEM):
        b, c = i % _B, i // _B
        if i + 2 < _NITEM:
            if i >= 2:
                store_copy(i - 2).wait()
            gather_copy(i + 2).start()
        if b == 0 and c + 1 < _NCH:
            pe_copy(c + 1).start()
        if b == 0:
            pe_copy(c).wait()
        gather_copy(i).wait()

        rb, pb = rows[b], pes[c % 2]

        def body_r(r, _):
            def body_j(j, _):
                for jj in range(4):
                    sl = pl.ds((j * 4 + jj) * _LANES, _LANES)
                    rb[r, sl] = rb[r, sl] * _SCALE + pb[r, sl]
                return 0

            return lax.fori_loop(0, _D // (_LANES * 4), body_j, 0)

        lax.fori_loop(0, _PC, body_r, 0)
        store_copy(i).start()

    for i in range(_NITEM - 4, _NITEM):
        store_copy(i).wait()


def kernel(token_ids, W):
    ids = token_ids.astype(jnp.int32)
    pe = jnp.asarray(_PE)
    return _emb_kernel(ids, W, pe)
